# baseline (speedup 1.0000x reference)
import functools

import jax
import jax.numpy as jnp
from jax import lax
from jax.experimental import pallas as pl
from jax.experimental.pallas import tpu as pltpu
from jax.experimental.pallas import tpu_sc as plsc

VOCAB_SIZE = 40000
EMBED_DIM = 8
SEQ_LEN = 428
SEQ_PAD = 512


@functools.cache
def _build_gather():
    info = plsc.get_sparse_core_info()
    nc, ns = info.num_cores, info.num_subcores
    nw = nc * ns
    bpw = SEQ_PAD // nw
    mesh = plsc.VectorSubcoreMesh(core_axis_name="c", subcore_axis_name="s")

    @functools.partial(
        pl.kernel,
        mesh=mesh,
        out_type=jax.ShapeDtypeStruct((SEQ_PAD, EMBED_DIM), jnp.float32),
        scratch_types=[
            pltpu.VMEM((bpw,), jnp.int32),
            pltpu.VMEM((bpw, EMBED_DIM), jnp.float32),
            pltpu.SemaphoreType.DMA,
        ],
        compiler_params=pltpu.CompilerParams(use_tc_tiling_on_sc=False),
    )
    def gather_sc(ids_hbm, table_hbm, out_hbm, idx_v, rows_v, sem):
        wid = lax.axis_index("s") * nc + lax.axis_index("c")
        base = wid * bpw
        pltpu.sync_copy(ids_hbm.at[pl.ds(base, bpw)], idx_v)
        pltpu.async_copy(table_hbm.at[idx_v], rows_v, sem).wait()
        pltpu.sync_copy(rows_v, out_hbm.at[pl.ds(base, bpw)])

    return gather_sc


def kernel(input_ids, table):
    ids = jnp.pad(input_ids.astype(jnp.int32), (0, SEQ_PAD - SEQ_LEN))
    out = _build_gather()(ids, table)
    return out[:SEQ_LEN].reshape(SEQ_LEN * EMBED_DIM)


# speedup vs baseline: 1.3800x; 1.3800x over previous
import functools

import jax
import jax.numpy as jnp
from jax import lax
from jax.experimental import pallas as pl
from jax.experimental.pallas import tpu as pltpu
from jax.experimental.pallas import tpu_sc as plsc

VOCAB_SIZE = 40000
EMBED_DIM = 8
SEQ_LEN = 428
SEQ_PAD = 512


@functools.cache
def _build_gather():
    info = plsc.get_sparse_core_info()
    nc, ns = info.num_cores, info.num_subcores
    nw = nc * ns
    bpw = SEQ_PAD // nw
    opw = bpw * EMBED_DIM
    mesh = plsc.VectorSubcoreMesh(core_axis_name="c", subcore_axis_name="s")

    @functools.partial(
        pl.kernel,
        mesh=mesh,
        out_type=jax.ShapeDtypeStruct((SEQ_PAD * EMBED_DIM,), jnp.float32),
        scratch_types=[
            pltpu.VMEM((bpw,), jnp.int32),
            pltpu.VMEM((bpw, 8, EMBED_DIM), jnp.float32),
            pltpu.VMEM((opw,), jnp.float32),
            pltpu.SemaphoreType.DMA,
        ],
        compiler_params=pltpu.CompilerParams(needs_layout_passes=False),
    )
    def gather_sc(ids_hbm, table_hbm, out_hbm, idx_v, slabs_v, rows_v, sem):
        wid = lax.axis_index("s") * nc + lax.axis_index("c")
        base = wid * bpw
        pltpu.sync_copy(ids_hbm.at[pl.ds(base, bpw)], idx_v)
        ids = idx_v[...]
        slab_base = lax.bitwise_and(ids, -8)
        lane = lax.iota(jnp.int32, 16)
        copies = []
        for i in range(bpw):
            sb = pl.multiple_of(jnp.sum(jnp.where(lane == i, slab_base, 0)), 8)
            copies.append(
                pltpu.async_copy(
                    table_hbm.at[pl.ds(sb, 8), :], slabs_v.at[i], sem
                )
            )
        for cp in copies:
            cp.wait()
        c = lax.bitwise_and(lane, 7)
        half = lax.shift_right_logical(lane, 3)
        for g in range(opw // 16):
            r = half + 2 * g
            srow = lax.bitwise_and(plsc.load_gather(idx_v, [r]), 7)
            rows_v[pl.ds(g * 16, 16)] = plsc.load_gather(slabs_v, [r, srow, c])
        pltpu.sync_copy(rows_v, out_hbm.at[pl.ds(base * EMBED_DIM, opw)])

    return gather_sc


def kernel(input_ids, table):
    ids = jnp.pad(input_ids.astype(jnp.int32), (0, SEQ_PAD - SEQ_LEN))
    out = _build_gather()(ids, table)
    return out[: SEQ_LEN * EMBED_DIM]
